# baseline (device time: 198374 ns/iter reference)
import jax
import jax.numpy as jnp
from jax import lax
from jax.experimental import pallas as pl
from jax.experimental.pallas import tpu as pltpu

N_DEV = 32
N = 2048
STREAMS = ((+1, 0, 512), (-1, 1024, 512), (+1, 512, 512), (-1, 1536, 512))
N_STREAMS = len(STREAMS)


def kernel(x, w_mat):
    k_full, k_per = x.shape
    _, n = w_mat.shape
    m_per = k_full // N_DEV

    def body(x_ref, w_ref, out_ref, *scratch):
        send_bufs = scratch[0:N_STREAMS]
        recv_bufs = scratch[N_STREAMS:2 * N_STREAMS]
        send_sems = scratch[2 * N_STREAMS]
        recv_sems = scratch[2 * N_STREAMS + 1]
        credit_sems = scratch[2 * N_STREAMS + 2]
        amax_send = scratch[2 * N_STREAMS + 3]
        amax_recv = scratch[2 * N_STREAMS + 4]
        amax_send_sems = scratch[2 * N_STREAMS + 5]
        amax_recv_sems = scratch[2 * N_STREAMS + 6]
        p_ref = scratch[2 * N_STREAMS + 7]

        j = lax.axis_index("i")

        def l_of_k(k):
            k = jnp.mod(k, N_DEV)
            x = jnp.where(k < 16, 0, 1)
            kk = jnp.where(k < 16, k, 31 - k)
            z = kk // 4
            y4 = kk % 4
            y = jnp.where(z % 2 == 0, y4, 3 - y4)
            return 8 * z + 2 * y + jnp.where(y % 2 == 0, x, 1 - x)

        def k_of_l(l):
            z = l // 8
            r = l % 8
            y = r // 2
            t = r % 2
            x = jnp.where(y % 2 == 0, t, 1 - t)
            kk = 4 * z + jnp.where(z % 2 == 0, y, 3 - y)
            return jnp.where(x == 0, kk, 31 - kk)

        my_k = k_of_l(j)
        right = l_of_k(my_k + 1)
        left = l_of_k(my_k - 1)

        barrier_sem = pltpu.get_barrier_semaphore()
        for nbr in [left, right]:
            pl.semaphore_signal(
                barrier_sem, inc=1,
                device_id=(nbr,), device_id_type=pl.DeviceIdType.MESH,
            )
        pl.semaphore_wait(barrier_sem, 2)

        p_ref[...] = jnp.dot(
            x_ref[...], w_ref[...], preferred_element_type=jnp.float32
        )

        def partial(c, col0, ncols):
            return p_ref[pl.ds(c * m_per, m_per), col0:col0 + ncols]

        def stream_peers(d):
            return (right, left) if d == +1 else (left, right)

        def chunk_at(d, s):
            return l_of_k(my_k - d * (1 + s))

        def ring_rdma(si, slot, peer=None):
            d, _, _ = STREAMS[si]
            down, up = stream_peers(d)
            return pltpu.make_async_remote_copy(
                src_ref=send_bufs[si].at[slot],
                dst_ref=recv_bufs[si].at[slot],
                send_sem=send_sems.at[si, slot],
                recv_sem=recv_sems.at[si, slot],
                device_id=(down if peer is None else peer,),
                device_id_type=pl.DeviceIdType.MESH,
            )

        for s in range(N_DEV - 1):
            slot = s % 2
            for si in range(N_STREAMS):
                d, col0, ncols = STREAMS[si]
                down, up = stream_peers(d)
                if s >= 1:
                    ring_rdma(si, (s - 1) % 2).wait()
                p = partial(chunk_at(d, s), col0, ncols)
                if s == 0:
                    send_bufs[si][slot] = p
                else:
                    send_bufs[si][slot] = recv_bufs[si][(s - 1) % 2] + p
                    if s <= N_DEV - 3:
                        pl.semaphore_signal(
                            credit_sems.at[si], inc=1,
                            device_id=(up,),
                            device_id_type=pl.DeviceIdType.MESH,
                        )
                if s >= 2:
                    pl.semaphore_wait(credit_sems.at[si], 1)
                ring_rdma(si, slot).start()

        last = (N_DEV - 2) % 2
        parts = {}
        for si in range(N_STREAMS):
            d, col0, ncols = STREAMS[si]
            ring_rdma(si, last).wait()
            parts[col0] = recv_bufs[si][last] + partial(j, col0, ncols)
        y = jnp.maximum(
            jnp.concatenate([parts[c] for c in sorted(parts)], axis=1), 0.0
        )

        local_max = jnp.max(y)
        amax_send[...] = jnp.full((8, 128), local_max, jnp.float32)
        amax_recv[j] = amax_send[...]
        descs = []
        for k in range(1, N_DEV):
            tgt = jnp.mod(j + k, N_DEV)
            dsc = pltpu.make_async_remote_copy(
                src_ref=amax_send,
                dst_ref=amax_recv.at[j],
                send_sem=amax_send_sems.at[k - 1],
                recv_sem=amax_recv_sems.at[j],
                device_id=(tgt,),
                device_id_type=pl.DeviceIdType.MESH,
            )
            dsc.start()
            descs.append(dsc)
        for dsc in descs:
            dsc.wait_send()
        for k in range(1, N_DEV):
            src = jnp.mod(j + k, N_DEV)
            pltpu.make_async_remote_copy(
                src_ref=amax_send,
                dst_ref=amax_recv.at[src],
                send_sem=amax_send_sems.at[k - 1],
                recv_sem=amax_recv_sems.at[src],
                device_id=(src,),
                device_id_type=pl.DeviceIdType.MESH,
            ).wait_recv()
        amax_g = jnp.max(amax_recv[...])

        scale = amax_g / 448.0
        q = (y / scale).astype(jnp.float8_e4m3fn).astype(jnp.float32)
        out_ref[...] = q * scale

    scratch_shapes = (
        [pltpu.VMEM((2, m_per, w), jnp.float32) for _, _, w in STREAMS]
        + [pltpu.VMEM((2, m_per, w), jnp.float32) for _, _, w in STREAMS]
        + [
            pltpu.SemaphoreType.DMA((N_STREAMS, 2)),
            pltpu.SemaphoreType.DMA((N_STREAMS, 2)),
            pltpu.SemaphoreType.REGULAR((N_STREAMS,)),
            pltpu.VMEM((8, 128), jnp.float32),
            pltpu.VMEM((N_DEV, 8, 128), jnp.float32),
            pltpu.SemaphoreType.DMA((N_DEV - 1,)),
            pltpu.SemaphoreType.DMA((N_DEV,)),
            pltpu.VMEM((k_full, n), jnp.float32),
        ]
    )
    return pl.pallas_call(
        body,
        out_shape=jax.ShapeDtypeStruct((m_per, n), jnp.float32),
        in_specs=[
            pl.BlockSpec(memory_space=pltpu.VMEM),
            pl.BlockSpec(memory_space=pltpu.VMEM),
        ],
        out_specs=pl.BlockSpec(memory_space=pltpu.VMEM),
        scratch_shapes=scratch_shapes,
        compiler_params=pltpu.CompilerParams(
            collective_id=0, vmem_limit_bytes=100 * 1024 * 1024
        ),
    )(x, w_mat)


# device time: 193798 ns/iter; 1.0236x vs baseline; 1.0236x over previous
import jax
import jax.numpy as jnp
from jax import lax
from jax.experimental import pallas as pl
from jax.experimental.pallas import tpu as pltpu

N_DEV = 32
N = 2048
STREAMS = ((+1, 0, 512), (-1, 1024, 512), (+1, 512, 512), (-1, 1536, 512))
N_STREAMS = len(STREAMS)


def kernel(x, w_mat):
    k_full, k_per = x.shape
    _, n = w_mat.shape
    m_per = k_full // N_DEV

    def body(x_ref, w_ref, out_ref, *scratch):
        send_bufs = scratch[0:N_STREAMS]
        recv_bufs = scratch[N_STREAMS:2 * N_STREAMS]
        send_sems = scratch[2 * N_STREAMS]
        recv_sems = scratch[2 * N_STREAMS + 1]
        credit_sems = scratch[2 * N_STREAMS + 2]
        amax_send = scratch[2 * N_STREAMS + 3]
        amax_recv = scratch[2 * N_STREAMS + 4]
        amax_send_sems = scratch[2 * N_STREAMS + 5]
        amax_recv_sems = scratch[2 * N_STREAMS + 6]

        j = lax.axis_index("i")

        def l_of_k(k):
            k = jnp.mod(k, N_DEV)
            x = jnp.where(k < 16, 0, 1)
            kk = jnp.where(k < 16, k, 31 - k)
            z = kk // 4
            y4 = kk % 4
            y = jnp.where(z % 2 == 0, y4, 3 - y4)
            return 8 * z + 2 * y + jnp.where(y % 2 == 0, x, 1 - x)

        def k_of_l(l):
            z = l // 8
            r = l % 8
            y = r // 2
            t = r % 2
            x = jnp.where(y % 2 == 0, t, 1 - t)
            kk = 4 * z + jnp.where(z % 2 == 0, y, 3 - y)
            return jnp.where(x == 0, kk, 31 - kk)

        my_k = k_of_l(j)
        right = l_of_k(my_k + 1)
        left = l_of_k(my_k - 1)

        barrier_sem = pltpu.get_barrier_semaphore()
        for nbr in [left, right]:
            pl.semaphore_signal(
                barrier_sem, inc=1,
                device_id=(nbr,), device_id_type=pl.DeviceIdType.MESH,
            )
        pl.semaphore_wait(barrier_sem, 2)

        def partial(c, col0, ncols):
            return jnp.dot(
                x_ref[pl.ds(c * m_per, m_per), :],
                w_ref[:, col0:col0 + ncols],
                preferred_element_type=jnp.float32,
            )

        def stream_peers(d):
            return (right, left) if d == +1 else (left, right)

        def chunk_at(d, s):
            return l_of_k(my_k - d * (1 + s))

        def ring_rdma(si, slot, peer=None):
            d, _, _ = STREAMS[si]
            down, up = stream_peers(d)
            return pltpu.make_async_remote_copy(
                src_ref=send_bufs[si].at[slot],
                dst_ref=recv_bufs[si].at[slot],
                send_sem=send_sems.at[si, slot],
                recv_sem=recv_sems.at[si, slot],
                device_id=(down if peer is None else peer,),
                device_id_type=pl.DeviceIdType.MESH,
            )

        for s in range(N_DEV - 1):
            slot = s % 2
            for si in range(N_STREAMS):
                d, col0, ncols = STREAMS[si]
                down, up = stream_peers(d)
                if s >= 1:
                    ring_rdma(si, (s - 1) % 2).wait()
                p = partial(chunk_at(d, s), col0, ncols)
                if s == 0:
                    send_bufs[si][slot] = p
                else:
                    send_bufs[si][slot] = recv_bufs[si][(s - 1) % 2] + p
                    if s <= N_DEV - 3:
                        pl.semaphore_signal(
                            credit_sems.at[si], inc=1,
                            device_id=(up,),
                            device_id_type=pl.DeviceIdType.MESH,
                        )
                if s >= 2:
                    pl.semaphore_wait(credit_sems.at[si], 1)
                ring_rdma(si, slot).start()

        last = (N_DEV - 2) % 2
        parts = {}
        for si in range(N_STREAMS):
            d, col0, ncols = STREAMS[si]
            ring_rdma(si, last).wait()
            parts[col0] = recv_bufs[si][last] + partial(j, col0, ncols)
        y = jnp.maximum(
            jnp.concatenate([parts[c] for c in sorted(parts)], axis=1), 0.0
        )

        local_max = jnp.max(y)
        amax_send[...] = jnp.full((8, 128), local_max, jnp.float32)
        amax_recv[j] = amax_send[...]
        descs = []
        for k in range(1, N_DEV):
            tgt = jnp.mod(j + k, N_DEV)
            dsc = pltpu.make_async_remote_copy(
                src_ref=amax_send,
                dst_ref=amax_recv.at[j],
                send_sem=amax_send_sems.at[k - 1],
                recv_sem=amax_recv_sems.at[j],
                device_id=(tgt,),
                device_id_type=pl.DeviceIdType.MESH,
            )
            dsc.start()
            descs.append(dsc)
        for dsc in descs:
            dsc.wait_send()
        for k in range(1, N_DEV):
            src = jnp.mod(j + k, N_DEV)
            pltpu.make_async_remote_copy(
                src_ref=amax_send,
                dst_ref=amax_recv.at[src],
                send_sem=amax_send_sems.at[k - 1],
                recv_sem=amax_recv_sems.at[src],
                device_id=(src,),
                device_id_type=pl.DeviceIdType.MESH,
            ).wait_recv()
        amax_g = jnp.max(amax_recv[...])

        scale = amax_g / 448.0
        q = (y / scale).astype(jnp.float8_e4m3fn).astype(jnp.float32)
        out_ref[...] = q * scale

    scratch_shapes = (
        [pltpu.VMEM((2, m_per, w), jnp.float32) for _, _, w in STREAMS]
        + [pltpu.VMEM((2, m_per, w), jnp.float32) for _, _, w in STREAMS]
        + [
            pltpu.SemaphoreType.DMA((N_STREAMS, 2)),
            pltpu.SemaphoreType.DMA((N_STREAMS, 2)),
            pltpu.SemaphoreType.REGULAR((N_STREAMS,)),
            pltpu.VMEM((8, 128), jnp.float32),
            pltpu.VMEM((N_DEV, 8, 128), jnp.float32),
            pltpu.SemaphoreType.DMA((N_DEV - 1,)),
            pltpu.SemaphoreType.DMA((N_DEV,)),
        ]
    )
    return pl.pallas_call(
        body,
        out_shape=jax.ShapeDtypeStruct((m_per, n), jnp.float32),
        in_specs=[
            pl.BlockSpec(memory_space=pltpu.VMEM),
            pl.BlockSpec(memory_space=pltpu.VMEM),
        ],
        out_specs=pl.BlockSpec(memory_space=pltpu.VMEM),
        scratch_shapes=scratch_shapes,
        compiler_params=pltpu.CompilerParams(
            collective_id=0, vmem_limit_bytes=100 * 1024 * 1024
        ),
    )(x, w_mat)
